# baseline (speedup 1.0000x reference)
import functools

import jax
import jax.numpy as jnp
from jax import lax
from jax.experimental import pallas as pl
from jax.experimental.pallas import tpu as pltpu
from jax.experimental.pallas import tpu_sc as plsc

B = 4096
L = 200
D = 32
VOCAB = 1000000
ROWS = B * L
NW = 32
TOK = 1024
NITEM = ROWS // TOK
IPW = NITEM // NW
HALF = TOK // 2
STAGE = 4 * 32 * 128

_mesh = plsc.VectorSubcoreMesh(core_axis_name="c", subcore_axis_name="s")


@functools.partial(
    pl.kernel,
    mesh=_mesh,
    out_type=jax.ShapeDtypeStruct((ROWS * D,), jnp.float32),
    compiler_params=pltpu.CompilerParams(
        use_tc_tiling_on_sc=False, needs_layout_passes=False),
    scratch_types=[
        pltpu.VMEM((TOK,), jnp.int32),
        pltpu.VMEM((TOK,), jnp.int32),
        pltpu.VMEM((TOK, D), jnp.float32),
        pltpu.VMEM((TOK, D), jnp.float32),
        pltpu.VMEM((STAGE,), jnp.float32),
        pltpu.VMEM((STAGE,), jnp.float32),
        pltpu.VMEM((L, D), jnp.float32),
        pltpu.SemaphoreType.DMA,
        pltpu.SemaphoreType.DMA,
        pltpu.SemaphoreType.DMA,
        pltpu.SemaphoreType.DMA,
    ],
)
def _emb_lookup(idx_hbm, tok_hbm, pos_hbm, out_hbm,
                idx0, idx1, g0, g1, st0, st1, pos_v,
                gsem0, gsem1, ssem0, ssem1):
    w = lax.axis_index("s") * 2 + lax.axis_index("c")
    item0 = w * IPW

    pltpu.sync_copy(pos_hbm, pos_v)

    idx_bufs = (idx0, idx1)
    g_bufs = (g0, g1)
    stages = (st0, st1)
    ssems = (ssem0, ssem1)
    gsems = (gsem0, gsem1)

    iota = jax.lax.iota(jnp.int32, 16)
    pat_lo = (iota // 8) * 4096 + lax.rem(iota, 8) * 128
    pat_hi = pat_lo + 2 * 4096

    def item_offsets(k):
        item = item0 + k
        l_s = item // 4
        q_s = lax.rem(item, 4)
        return l_s, q_s, l_s * B + q_s * TOK

    def store_tiles(stage, l_s, q_s, blk, ssem):
        for dg in range(4):
            dst = out_hbm.at[pl.ds(
                (l_s * 32 + dg * 8 + q_s * 2 + blk) * 4096, 4096)]
            pltpu.async_copy(stage.at[pl.ds(dg * 4096, 4096)], dst, ssem)

    def drain_tiles(stage, ssem):
        for dg in range(4):
            pltpu.make_async_copy(
                stage.at[pl.ds(dg * 4096, 4096)],
                out_hbm.at[pl.ds(0, 4096)], ssem).wait()

    for p in range(2):
        _, _, off = item_offsets(p)
        pltpu.sync_copy(idx_hbm.at[pl.ds(off, TOK)], idx_bufs[p])
        pltpu.async_copy(tok_hbm.at[idx_bufs[p]], g_bufs[p], gsems[p])

    step = 0
    for k in range(IPW):
        bsel = k % 2
        idx_v = idx_bufs[bsel]
        g = g_bufs[bsel]
        l_s, q_s, _ = item_offsets(k)
        plo = pos_v[l_s, pl.ds(0, 16)]
        phi = pos_v[l_s, pl.ds(16, 16)]

        pltpu.make_async_copy(tok_hbm.at[idx_v], g, gsems[bsel]).wait()

        for blk in range(2):
            psel = step % 2
            stage = stages[psel]
            if step >= 2:
                drain_tiles(stage, ssems[psel])

            @plsc.parallel_loop(0, HALF, unroll=16)
            def tok_body(rr):
                base = (rr // 128) * 1024 + lax.rem(rr, 128)
                r = blk * HALF + rr
                cvec = jnp.full((16,), base, jnp.int32)
                vlo = g[r, pl.ds(0, 16)] + plo
                vhi = g[r, pl.ds(16, 16)] + phi
                plsc.store_scatter(stage, [pat_lo + cvec], vlo)
                plsc.store_scatter(stage, [pat_hi + cvec], vhi)
            store_tiles(stage, l_s, q_s, blk, ssems[psel])
            step += 1

        nxt = k + 2
        if nxt < IPW:
            _, _, off = item_offsets(nxt)
            pltpu.sync_copy(idx_hbm.at[pl.ds(off, TOK)], idx_v)
            pltpu.async_copy(tok_hbm.at[idx_v], g, gsems[bsel])

    for psel in range(2):
        drain_tiles(stages[psel], ssems[psel])


def kernel(inputs, tok_table, pos_table):
    idx_lmaj = inputs.T.reshape(ROWS).astype(jnp.int32)
    out = _emb_lookup(idx_lmaj, tok_table, pos_table)
    a6 = out.reshape(L, 4, 8, 4, 8, 128)
    return a6.transpose(2, 3, 5, 0, 1, 4).reshape(B, L, D)


# speedup vs baseline: 1.0037x; 1.0037x over previous
import functools

import jax
import jax.numpy as jnp
from jax import lax
from jax.experimental import pallas as pl
from jax.experimental.pallas import tpu as pltpu
from jax.experimental.pallas import tpu_sc as plsc

B = 4096
L = 200
D = 32
VOCAB = 1000000
ROWS = B * L
NW = 32
TOK = 1024
NITEM = ROWS // TOK
IPW = NITEM // NW
HALF = TOK // 2
STAGE = 4 * 32 * 128

_mesh = plsc.VectorSubcoreMesh(core_axis_name="c", subcore_axis_name="s")


@functools.partial(
    pl.kernel,
    mesh=_mesh,
    out_type=jax.ShapeDtypeStruct((ROWS * D,), jnp.float32),
    compiler_params=pltpu.CompilerParams(
        use_tc_tiling_on_sc=False, needs_layout_passes=False),
    scratch_types=[
        pltpu.VMEM((TOK,), jnp.int32),
        pltpu.VMEM((TOK,), jnp.int32),
        pltpu.VMEM((TOK, D), jnp.float32),
        pltpu.VMEM((TOK, D), jnp.float32),
        pltpu.VMEM((STAGE,), jnp.float32),
        pltpu.VMEM((STAGE,), jnp.float32),
        pltpu.VMEM((L, D), jnp.float32),
        pltpu.SemaphoreType.DMA,
        pltpu.SemaphoreType.DMA,
        pltpu.SemaphoreType.DMA,
        pltpu.SemaphoreType.DMA,
    ],
)
def _emb_lookup(idx_hbm, tok_hbm, pos_hbm, out_hbm,
                idx0, idx1, g0, g1, st0, st1, pos_v,
                gsem0, gsem1, ssem0, ssem1):
    w = lax.axis_index("s") * 2 + lax.axis_index("c")
    item0 = w * IPW

    pltpu.sync_copy(pos_hbm, pos_v)

    idx_bufs = (idx0, idx1)
    g_bufs = (g0, g1)
    stages = (st0, st1)
    ssems = (ssem0, ssem1)
    gsems = (gsem0, gsem1)

    iota = jax.lax.iota(jnp.int32, 16)
    pat_lo = (iota // 8) * 4096 + lax.rem(iota, 8) * 128
    pat_hi = pat_lo + 2 * 4096

    def item_offsets(k):
        item = item0 + k
        l_s = item // 4
        q_s = lax.rem(item, 4)
        return l_s, q_s, l_s * B + q_s * TOK

    def store_tiles(stage, l_s, q_s, blk, ssem):
        for dg in range(4):
            dst = out_hbm.at[pl.ds(
                (l_s * 32 + dg * 8 + q_s * 2 + blk) * 4096, 4096)]
            pltpu.async_copy(stage.at[pl.ds(dg * 4096, 4096)], dst, ssem)

    def drain_tiles(stage, ssem):
        for dg in range(4):
            pltpu.make_async_copy(
                stage.at[pl.ds(dg * 4096, 4096)],
                out_hbm.at[pl.ds(0, 4096)], ssem).wait()

    for p in range(2):
        _, _, off = item_offsets(p)
        pltpu.sync_copy(idx_hbm.at[pl.ds(off, TOK)], idx_bufs[p])
        pltpu.async_copy(tok_hbm.at[idx_bufs[p]], g_bufs[p], gsems[p])

    step = 0
    for k in range(IPW):
        bsel = k % 2
        idx_v = idx_bufs[bsel]
        g = g_bufs[bsel]
        l_s, q_s, _ = item_offsets(k)
        plo = pos_v[l_s, pl.ds(0, 16)]
        phi = pos_v[l_s, pl.ds(16, 16)]

        pltpu.make_async_copy(tok_hbm.at[idx_v], g, gsems[bsel]).wait()

        for blk in range(2):
            psel = step % 2
            stage = stages[psel]
            if step >= 2:
                drain_tiles(stage, ssems[psel])

            @plsc.parallel_loop(0, HALF, unroll=8)
            def tok_body(rr):
                base = (rr // 128) * 1024 + lax.rem(rr, 128)
                r = blk * HALF + rr
                cvec = jnp.full((16,), base, jnp.int32)
                vlo = g[r, pl.ds(0, 16)] + plo
                vhi = g[r, pl.ds(16, 16)] + phi
                plsc.store_scatter(stage, [pat_lo + cvec], vlo)
                plsc.store_scatter(stage, [pat_hi + cvec], vhi)
            store_tiles(stage, l_s, q_s, blk, ssems[psel])
            step += 1

        nxt = k + 2
        if nxt < IPW:
            _, _, off = item_offsets(nxt)
            pltpu.sync_copy(idx_hbm.at[pl.ds(off, TOK)], idx_v)
            pltpu.async_copy(tok_hbm.at[idx_v], g, gsems[bsel])

    for psel in range(2):
        drain_tiles(stages[psel], ssems[psel])


def kernel(inputs, tok_table, pos_table):
    idx_lmaj = inputs.T.reshape(ROWS).astype(jnp.int32)
    out = _emb_lookup(idx_lmaj, tok_table, pos_table)
    a6 = out.reshape(L, 4, 8, 4, 8, 128)
    return a6.transpose(2, 3, 5, 0, 1, 4).reshape(B, L, D)
